# baseline (speedup 1.0000x reference)
import functools

import jax
import jax.numpy as jnp
from jax import lax
from jax.experimental import pallas as pl
from jax.experimental.pallas import tpu as pltpu
from jax.experimental.pallas import tpu_sc as plsc

VOCAB = 1000000
EMBED = 64
B = 16384
K = 20

NC = 2
NS = 16
L = 16
NW = NC * NS
BPW = B // NW
SB = 32
NSUB = BPW // SB
CTX_SB = SB * K
IDX_CHUNK = 128


def _fire_gather(table, idx_ref, idx_off, dst_ref, n_rows, sem):
    cps = []
    for j in range(0, n_rows, IDX_CHUNK):
        c = min(IDX_CHUNK, n_rows - j)
        cps.append(
            pltpu.async_copy(
                table.at[idx_ref.at[pl.ds(idx_off + j, c)]],
                dst_ref.at[pl.ds(j, c)],
                sem,
            )
        )
    return cps


def _sc_body(ctr_idx_hbm, ctx_idx_hbm, ctr_tab, ctx_tab, out_hbm,
             ctr_idx_v, ctx_idx_v, ctr_rows, ctx_buf0, ctx_buf1, out_v,
             sem_c, sem0, sem1):
    wid = lax.axis_index("s") * NC + lax.axis_index("c")
    b0 = wid * BPW

    pltpu.sync_copy(ctr_idx_hbm.at[pl.ds(b0, BPW)], ctr_idx_v)
    pltpu.sync_copy(ctx_idx_hbm.at[pl.ds(b0 * K, BPW * K)], ctx_idx_v)

    ctr_cps = _fire_gather(ctr_tab, ctr_idx_v, 0, ctr_rows, BPW, sem_c)
    bufs = (ctx_buf0, ctx_buf1)
    sems = (sem0, sem1)
    cps = [None, None]
    cps[0] = _fire_gather(ctx_tab, ctx_idx_v, 0, ctx_buf0, CTX_SB, sem0)
    for d in ctr_cps:
        d.wait()

    lanes = lax.iota(jnp.int32, L)
    zero = jnp.zeros((L,), jnp.float32)

    for s in range(NSUB):
        cur = s % 2
        if s + 1 < NSUB:
            nxt = (s + 1) % 2
            cps[nxt] = _fire_gather(
                ctx_tab, ctx_idx_v, (s + 1) * CTX_SB, bufs[nxt], CTX_SB,
                sems[nxt])
        for d in cps[cur]:
            d.wait()
        buf = bufs[cur]

        for g in range(SB // L):
            row_ctr = s * SB + g * L + lanes
            rowbase = (g * L + lanes) * K

            def e_body(e, accs, row_ctr=row_ctr, rowbase=rowbase, buf=buf):
                col = jnp.full((L,), e, jnp.int32)
                ctr = plsc.load_gather(ctr_rows, [row_ctr, col])
                new = []
                for k in range(K):
                    v = plsc.load_gather(buf, [rowbase + k, col])
                    new.append(accs[k] + ctr * v)
                return tuple(new)

            accs = lax.fori_loop(0, EMBED, e_body, (zero,) * K)
            for k in range(K):
                plsc.store_scatter(out_v, [rowbase + k], accs[k])

        pltpu.sync_copy(out_v, out_hbm.at[pl.ds((b0 + s * SB) * K, CTX_SB)])


_mesh = plsc.VectorSubcoreMesh(core_axis_name="c", subcore_axis_name="s")

_sc_kernel = functools.partial(
    pl.kernel,
    out_type=jax.ShapeDtypeStruct((B * K,), jnp.float32),
    mesh=_mesh,
    scratch_types=[
        pltpu.VMEM((BPW,), jnp.int32),
        pltpu.VMEM((BPW * K,), jnp.int32),
        pltpu.VMEM((BPW, EMBED), jnp.float32),
        pltpu.VMEM((CTX_SB, EMBED), jnp.float32),
        pltpu.VMEM((CTX_SB, EMBED), jnp.float32),
        pltpu.VMEM((CTX_SB,), jnp.float32),
        pltpu.SemaphoreType.DMA,
        pltpu.SemaphoreType.DMA,
        pltpu.SemaphoreType.DMA,
    ],
    compiler_params=pltpu.CompilerParams(
        use_tc_tiling_on_sc=False, needs_layout_passes=False),
)(_sc_body)


def kernel(centers, context_negatives, center_table, context_table):
    ctr_idx = centers.reshape(B).astype(jnp.int32)
    ctx_idx = context_negatives.reshape(B * K).astype(jnp.int32)
    out = _sc_kernel(ctr_idx, ctx_idx, center_table, context_table)
    return out.reshape(B, K)


# speedup vs baseline: 1.0025x; 1.0025x over previous
import functools

import jax
import jax.numpy as jnp
from jax import lax
from jax.experimental import pallas as pl
from jax.experimental.pallas import tpu as pltpu
from jax.experimental.pallas import tpu_sc as plsc

VOCAB = 1000000
EMBED = 64
B = 16384
K = 20

NC = 2
NS = 16
L = 16
NW = NC * NS
BPW = B // NW
SB = 32
NSUB = BPW // SB
CTX_SB = SB * K
IDX_CHUNK = 640


def _fire_gather(table, idx_ref, idx_off, dst_ref, n_rows, sem):
    cps = []
    for j in range(0, n_rows, IDX_CHUNK):
        c = min(IDX_CHUNK, n_rows - j)
        cps.append(
            pltpu.async_copy(
                table.at[idx_ref.at[pl.ds(idx_off + j, c)]],
                dst_ref.at[pl.ds(j, c)],
                sem,
            )
        )
    return cps


def _sc_body(ctr_idx_hbm, ctx_idx_hbm, ctr_tab, ctx_tab, out_hbm,
             ctr_idx_v, ctx_idx_v, ctr_rows, ctx_buf0, ctx_buf1, out_v,
             sem_c, sem0, sem1):
    wid = lax.axis_index("s") * NC + lax.axis_index("c")
    b0 = wid * BPW

    pltpu.sync_copy(ctr_idx_hbm.at[pl.ds(b0, BPW)], ctr_idx_v)
    pltpu.sync_copy(ctx_idx_hbm.at[pl.ds(b0 * K, BPW * K)], ctx_idx_v)

    ctr_cps = _fire_gather(ctr_tab, ctr_idx_v, 0, ctr_rows, BPW, sem_c)
    bufs = (ctx_buf0, ctx_buf1)
    sems = (sem0, sem1)
    cps = [None, None]
    cps[0] = _fire_gather(ctx_tab, ctx_idx_v, 0, ctx_buf0, CTX_SB, sem0)
    for d in ctr_cps:
        d.wait()

    lanes = lax.iota(jnp.int32, L)
    zero = jnp.zeros((L,), jnp.float32)

    for s in range(NSUB):
        cur = s % 2
        if s + 1 < NSUB:
            nxt = (s + 1) % 2
            cps[nxt] = _fire_gather(
                ctx_tab, ctx_idx_v, (s + 1) * CTX_SB, bufs[nxt], CTX_SB,
                sems[nxt])
        for d in cps[cur]:
            d.wait()
        buf = bufs[cur]

        for g in range(SB // L):
            row_ctr = s * SB + g * L + lanes
            rowbase = (g * L + lanes) * K

            def e_body(e, accs, row_ctr=row_ctr, rowbase=rowbase, buf=buf):
                col = jnp.full((L,), e, jnp.int32)
                ctr = plsc.load_gather(ctr_rows, [row_ctr, col])
                new = []
                for k in range(K):
                    v = plsc.load_gather(buf, [rowbase + k, col])
                    new.append(accs[k] + ctr * v)
                return tuple(new)

            accs = lax.fori_loop(0, EMBED, e_body, (zero,) * K)
            for k in range(K):
                plsc.store_scatter(out_v, [rowbase + k], accs[k])

        pltpu.sync_copy(out_v, out_hbm.at[pl.ds((b0 + s * SB) * K, CTX_SB)])


_mesh = plsc.VectorSubcoreMesh(core_axis_name="c", subcore_axis_name="s")

_sc_kernel = functools.partial(
    pl.kernel,
    out_type=jax.ShapeDtypeStruct((B * K,), jnp.float32),
    mesh=_mesh,
    scratch_types=[
        pltpu.VMEM((BPW,), jnp.int32),
        pltpu.VMEM((BPW * K,), jnp.int32),
        pltpu.VMEM((BPW, EMBED), jnp.float32),
        pltpu.VMEM((CTX_SB, EMBED), jnp.float32),
        pltpu.VMEM((CTX_SB, EMBED), jnp.float32),
        pltpu.VMEM((CTX_SB,), jnp.float32),
        pltpu.SemaphoreType.DMA,
        pltpu.SemaphoreType.DMA,
        pltpu.SemaphoreType.DMA,
    ],
    compiler_params=pltpu.CompilerParams(
        use_tc_tiling_on_sc=False, needs_layout_passes=False),
)(_sc_body)


def kernel(centers, context_negatives, center_table, context_table):
    ctr_idx = centers.reshape(B).astype(jnp.int32)
    ctx_idx = context_negatives.reshape(B * K).astype(jnp.int32)
    out = _sc_kernel(ctr_idx, ctx_idx, center_table, context_table)
    return out.reshape(B, K)
